# Initial kernel scaffold; baseline (speedup 1.0000x reference)
import jax
import jax.numpy as jnp
from jax.experimental import pallas as pl


def kernel(queries, keys, top_k):
    raise NotImplementedError("write your pallas kernel here")


# speedup vs baseline: 2.0690x; 2.0690x over previous
import functools

import jax
import jax.numpy as jnp
from jax.experimental import pallas as pl
from jax.experimental.pallas import tpu as pltpu

_B = 2048
_RUN = 128
_NEG = -3.0e38
_TOPK = 10


def _fused_topk_body(nb, kb, n_keys, q_ref, k_ref, vals_ref, idx_ref):
    b = pl.program_id(0)
    nq = q_ref.shape[0]

    @pl.when(b == 0)
    def _init():
        vals_ref[...] = jnp.full(vals_ref.shape, _NEG, jnp.float32)
        idx_ref[...] = jnp.zeros(idx_ref.shape, jnp.int32)

    q = q_ref[...]
    qn = q / jnp.sqrt(jnp.sum(q * q, axis=1, keepdims=True))
    k = k_ref[...]
    knorm = jnp.sqrt(jnp.sum(k * k, axis=1, keepdims=True))
    kn = k / jnp.where(knorm > 0.0, knorm, 1.0)

    s = jax.lax.dot_general(qn, kn, (((1,), (1,)), ((), ())),
                            preferred_element_type=jnp.float32)

    col = jax.lax.broadcasted_iota(jnp.int32, (1, kb), 1) + b * kb
    s = jnp.where(col < n_keys, s, _NEG)

    run_vals = vals_ref[...]
    run_idx = idx_ref[...]

    c = jnp.concatenate([s, run_vals], axis=1)
    iota_c = jax.lax.broadcasted_iota(jnp.int32, (nq, kb + _RUN), 1)
    lane_iota = jax.lax.broadcasted_iota(jnp.int32, (nq, _RUN), 1)
    big = 1 << 30

    for j in range(_TOPK):
        m = jnp.max(c, axis=1, keepdims=True)
        t = jnp.where(c == m, iota_c, big)
        pos = jnp.min(t, axis=1, keepdims=True)
        in_run = pos >= kb
        run_lane = pos - kb
        gi = jnp.sum(jnp.where(lane_iota == run_lane, run_idx, 0),
                     axis=1, keepdims=True)
        gidx = jnp.where(in_run, gi, b * kb + pos)
        vals_ref[:, j:j + 1] = m
        idx_ref[:, j:j + 1] = gidx
        c = jnp.where(iota_c == pos, _NEG, c)


def kernel(queries, keys, top_k):
    nq, d = queries.shape
    n_keys = keys.shape[0]
    kb = _B
    nb = (n_keys + kb - 1) // kb
    kpad = nb * kb
    kp = jnp.pad(keys, ((0, kpad - n_keys), (0, 0)))

    vals, idx = pl.pallas_call(
        functools.partial(_fused_topk_body, nb, kb, n_keys),
        grid=(nb,),
        in_specs=[
            pl.BlockSpec((nq, d), lambda b: (0, 0)),
            pl.BlockSpec((kb, d), lambda b: (b, 0)),
        ],
        out_specs=[
            pl.BlockSpec((nq, _RUN), lambda b: (0, 0)),
            pl.BlockSpec((nq, _RUN), lambda b: (0, 0)),
        ],
        out_shape=[
            jax.ShapeDtypeStruct((nq, _RUN), jnp.float32),
            jax.ShapeDtypeStruct((nq, _RUN), jnp.int32),
        ],
        compiler_params=pltpu.CompilerParams(
            dimension_semantics=("arbitrary",)),
    )(queries, kp)
    return vals[:, :_TOPK], idx[:, :_TOPK]


# speedup vs baseline: 3.7315x; 1.8035x over previous
import functools

import jax
import jax.numpy as jnp
from jax import lax
from jax.experimental import pallas as pl
from jax.experimental.pallas import tpu as pltpu
from jax.experimental.pallas import tpu_sc as plsc

_B = 2048
_G = 16
_NCAND = 160
_NEG = -3.0e38
_TOPK = 10


def _score_chunkmax_body(n_keys, q_ref, k_ref, cm_ref):
    b = pl.program_id(0)
    kb = k_ref.shape[0]

    qn = q_ref[...]
    kn = k_ref[...]
    s = lax.dot_general(qn, kn, (((1,), (1,)), ((), ())),
                        preferred_element_type=jnp.float32)
    col = lax.broadcasted_iota(jnp.int32, (1, kb), 1) + b * kb
    s = jnp.where(col < n_keys, s, _NEG)

    m = s[:, 0:128]
    for g in range(1, _G):
        m = jnp.maximum(m, s[:, g * 128:(g + 1) * 128])
    cm_ref[...] = m


def _chunk_topk_body(kb, cm_ref, kid_ref):
    c = cm_ref[...]
    t, w = c.shape
    iota_c = lax.broadcasted_iota(jnp.int32, (t, w), 1)
    giota = lax.broadcasted_iota(jnp.int32, (1, _G), 1)
    big = 1 << 30
    for j in range(_TOPK):
        m = jnp.max(c, axis=1, keepdims=True)
        pos = jnp.min(jnp.where(c == m, iota_c, big), axis=1, keepdims=True)
        blk = lax.shift_right_logical(pos, 7)
        lane = pos - blk * 128
        kid16 = blk * kb + lane + giota * 128
        kid_ref[:, j * _G:(j + 1) * _G] = kid16
        c = jnp.where(iota_c == pos, _NEG, c)


def _sc_gather(kn_pad, kid):
    nq = kid.shape[0]
    d = kn_pad.shape[1]
    mesh = plsc.VectorSubcoreMesh(core_axis_name="c", subcore_axis_name="s")
    nw = 32
    qpw = nq // nw
    half = _NCAND // 2

    @functools.partial(
        pl.kernel,
        mesh=mesh,
        out_type=jax.ShapeDtypeStruct((nq, _NCAND, d), jnp.float32),
        scratch_types=[
            pltpu.VMEM((_NCAND,), jnp.int32),
            pltpu.VMEM((_NCAND, d), jnp.float32),
            pltpu.SemaphoreType.DMA,
        ],
    )
    def gather_kernel(kn_hbm, kid_hbm, out_hbm, idb, rows, sem):
        wid = lax.axis_index("s") * 2 + lax.axis_index("c")
        base = wid * qpw

        def body(i, carry):
            qq = base + i
            pltpu.sync_copy(kid_hbm.at[qq], idb)
            d1 = pltpu.async_copy(
                kn_hbm.at[idb.at[pl.ds(0, half)]], rows.at[pl.ds(0, half)], sem)
            d2 = pltpu.async_copy(
                kn_hbm.at[idb.at[pl.ds(half, half)]], rows.at[pl.ds(half, half)],
                sem)
            d1.wait()
            d2.wait()
            pltpu.sync_copy(rows, out_hbm.at[qq])
            return carry

        lax.fori_loop(0, qpw, body, 0)

    return gather_kernel(kn_pad, kid)


def _rescore_body(g_ref, qn_ref, kid_ref, vals_ref, idx_ref):
    t = g_ref.shape[0]
    wamp = t * _NCAND
    g2 = g_ref[...].reshape(wamp, 128)
    qn = qn_ref[...]
    full = lax.dot_general(qn, g2, (((1,), (1,)), ((), ())),
                           preferred_element_type=jnp.float32)
    ri = lax.broadcasted_iota(jnp.int32, (t, wamp), 0)
    ci = lax.broadcasted_iota(jnp.int32, (t, wamp), 1)
    lo = ri * _NCAND
    in_band = (ci >= lo) & (ci < lo + _NCAND)
    s = jnp.where(in_band & (full > 0.0), full, _NEG)
    kid = kid_ref[...]
    iota_c = lax.broadcasted_iota(jnp.int32, (t, _NCAND), 1)
    rrow = lax.broadcasted_iota(jnp.int32, (t, 1), 0)
    big = 1 << 30
    vals_ref[...] = jnp.zeros(vals_ref.shape, jnp.float32)
    idx_ref[...] = jnp.zeros(idx_ref.shape, jnp.int32)
    for j in range(_TOPK):
        m = jnp.max(s, axis=1, keepdims=True)
        pos = jnp.min(jnp.where(s == m, ci, big), axis=1, keepdims=True)
        jl = pos - rrow * _NCAND
        gi = jnp.sum(jnp.where(iota_c == jl, kid, 0), axis=1, keepdims=True)
        vals_ref[:, j:j + 1] = m
        idx_ref[:, j:j + 1] = gi
        s = jnp.where(ci == pos, _NEG, s)


def kernel(queries, keys, top_k):
    nq, d = queries.shape
    n_keys = keys.shape[0]
    kb = _B
    nb = (n_keys + kb - 1) // kb
    kpad = nb * kb
    qn = queries / jnp.linalg.norm(queries, axis=1, keepdims=True)
    kn = keys / jnp.linalg.norm(keys, axis=1, keepdims=True)
    knp = jnp.pad(kn, ((0, kpad - n_keys), (0, 0)))
    w = nb * 128

    cm = pl.pallas_call(
        functools.partial(_score_chunkmax_body, n_keys),
        grid=(nb,),
        in_specs=[
            pl.BlockSpec((nq, d), lambda b: (0, 0)),
            pl.BlockSpec((kb, d), lambda b: (b, 0)),
        ],
        out_specs=pl.BlockSpec((nq, 128), lambda b: (0, b)),
        out_shape=jax.ShapeDtypeStruct((nq, w), jnp.float32),
        compiler_params=pltpu.CompilerParams(
            dimension_semantics=("arbitrary",)),
    )(qn, knp)

    qt = 256
    kid = pl.pallas_call(
        functools.partial(_chunk_topk_body, kb),
        grid=(nq // qt,),
        in_specs=[pl.BlockSpec((qt, w), lambda i: (i, 0))],
        out_specs=pl.BlockSpec((qt, _NCAND), lambda i: (i, 0)),
        out_shape=jax.ShapeDtypeStruct((nq, _NCAND), jnp.int32),
        compiler_params=pltpu.CompilerParams(
            dimension_semantics=("arbitrary",)),
    )(cm)

    gk = _sc_gather(knp, kid)

    rt = 8
    vals, idx = pl.pallas_call(
        _rescore_body,
        grid=(nq // rt,),
        in_specs=[
            pl.BlockSpec((rt, _NCAND, d), lambda i: (i, 0, 0)),
            pl.BlockSpec((rt, d), lambda i: (i, 0)),
            pl.BlockSpec((rt, _NCAND), lambda i: (i, 0)),
        ],
        out_specs=[
            pl.BlockSpec((rt, 16), lambda i: (i, 0)),
            pl.BlockSpec((rt, 16), lambda i: (i, 0)),
        ],
        out_shape=[
            jax.ShapeDtypeStruct((nq, 16), jnp.float32),
            jax.ShapeDtypeStruct((nq, 16), jnp.int32),
        ],
        compiler_params=pltpu.CompilerParams(
            dimension_semantics=("arbitrary",)),
    )(gk, qn, kid)

    return vals[:, :_TOPK], idx[:, :_TOPK]


# speedup vs baseline: 5.6836x; 1.5231x over previous
import functools

import jax
import jax.numpy as jnp
from jax import lax
from jax.experimental import pallas as pl
from jax.experimental.pallas import tpu as pltpu
from jax.experimental.pallas import tpu_sc as plsc

_B = 2048
_G = 16
_NCAND = 160
_NEG = -3.0e38
_TOPK = 10


def _score_chunkmax_body(n_keys, q_ref, k_ref, cm_ref):
    b = pl.program_id(0)
    kb = k_ref.shape[0]

    qn = q_ref[...]
    kn = k_ref[...]
    s = lax.dot_general(qn, kn, (((1,), (1,)), ((), ())),
                        preferred_element_type=jnp.float32)
    col = lax.broadcasted_iota(jnp.int32, (1, kb), 1) + b * kb
    s = jnp.where(col < n_keys, s, _NEG)

    m = s[:, 0:128]
    for g in range(1, _G):
        m = jnp.maximum(m, s[:, g * 128:(g + 1) * 128])
    cm_ref[...] = m


def _chunk_topk_body(kb, cm_ref, kid_ref):
    c = cm_ref[...]
    t, w = c.shape
    iota_c = lax.broadcasted_iota(jnp.int32, (t, w), 1)
    giota = lax.broadcasted_iota(jnp.int32, (1, _G), 1)
    big = 1 << 30
    for j in range(_TOPK):
        m = jnp.max(c, axis=1, keepdims=True)
        pos = jnp.min(jnp.where(c == m, iota_c, big), axis=1, keepdims=True)
        blk = lax.shift_right_logical(pos, 7)
        lane = pos - blk * 128
        kid16 = blk * kb + lane + giota * 128
        kid_ref[:, j * _G:(j + 1) * _G] = kid16
        c = jnp.where(iota_c == pos, _NEG, c)


def _sc_gather(kn_pad, kid):
    nq = kid.shape[0]
    d = kn_pad.shape[1]
    mesh = plsc.VectorSubcoreMesh(core_axis_name="c", subcore_axis_name="s")
    nw = 32
    qpw = nq // nw
    half = _NCAND // 2

    @functools.partial(
        pl.kernel,
        mesh=mesh,
        out_type=jax.ShapeDtypeStruct((nq, _NCAND, d), jnp.float32),
        scratch_types=[
            pltpu.VMEM((_NCAND,), jnp.int32),
            pltpu.VMEM((_NCAND, d), jnp.float32),
            pltpu.SemaphoreType.DMA,
        ],
    )
    def gather_kernel(kn_hbm, kid_hbm, out_hbm, idb, rows, sem):
        wid = lax.axis_index("s") * 2 + lax.axis_index("c")
        base = wid * qpw

        def body(i, carry):
            qq = base + i
            pltpu.sync_copy(kid_hbm.at[qq], idb)
            d1 = pltpu.async_copy(
                kn_hbm.at[idb.at[pl.ds(0, half)]], rows.at[pl.ds(0, half)], sem)
            d2 = pltpu.async_copy(
                kn_hbm.at[idb.at[pl.ds(half, half)]], rows.at[pl.ds(half, half)],
                sem)
            d1.wait()
            d2.wait()
            pltpu.sync_copy(rows, out_hbm.at[qq])
            return carry

        lax.fori_loop(0, qpw, body, 0)

    return gather_kernel(kn_pad, kid)


def _rescore_body(g_ref, qn_ref, kid_ref, vals_ref, idx_ref):
    t = g_ref.shape[0]
    ncp = 256
    g = g_ref[...]
    gz = jnp.concatenate(
        [g, jnp.zeros((t, ncp - _NCAND, 128), jnp.float32)], axis=1)
    g2 = gz.reshape(t * ncp, 128)
    qn = qn_ref[...]
    full = lax.dot_general(qn, g2, (((1,), (1,)), ((), ())),
                           preferred_element_type=jnp.float32)
    s = jnp.concatenate(
        [full[i:i + 1, i * ncp:(i + 1) * ncp] for i in range(t)], axis=0)
    s = jnp.where(s > 0.0, s, _NEG)
    kid = kid_ref[...]
    iota_nc = lax.broadcasted_iota(jnp.int32, (t, _NCAND), 1)
    iota_p = lax.broadcasted_iota(jnp.int32, (t, ncp), 1)
    big = 1 << 30
    vals_ref[...] = jnp.zeros(vals_ref.shape, jnp.float32)
    idx_ref[...] = jnp.zeros(idx_ref.shape, jnp.int32)
    for j in range(_TOPK):
        m = jnp.max(s, axis=1, keepdims=True)
        pos = jnp.min(jnp.where(s == m, iota_p, big), axis=1, keepdims=True)
        gi = jnp.sum(jnp.where(iota_nc == pos, kid, 0), axis=1, keepdims=True)
        vals_ref[:, j:j + 1] = m
        idx_ref[:, j:j + 1] = gi
        s = jnp.where(iota_p == pos, _NEG, s)


def kernel(queries, keys, top_k):
    nq, d = queries.shape
    n_keys = keys.shape[0]
    kb = _B
    nb = (n_keys + kb - 1) // kb
    kpad = nb * kb
    qn = queries / jnp.linalg.norm(queries, axis=1, keepdims=True)
    kn = keys / jnp.linalg.norm(keys, axis=1, keepdims=True)
    knp = jnp.pad(kn, ((0, kpad - n_keys), (0, 0)))
    w = nb * 128

    cm = pl.pallas_call(
        functools.partial(_score_chunkmax_body, n_keys),
        grid=(nb,),
        in_specs=[
            pl.BlockSpec((nq, d), lambda b: (0, 0)),
            pl.BlockSpec((kb, d), lambda b: (b, 0)),
        ],
        out_specs=pl.BlockSpec((nq, 128), lambda b: (0, b)),
        out_shape=jax.ShapeDtypeStruct((nq, w), jnp.float32),
        compiler_params=pltpu.CompilerParams(
            dimension_semantics=("arbitrary",)),
    )(qn, knp)

    qt = 256
    kid = pl.pallas_call(
        functools.partial(_chunk_topk_body, kb),
        grid=(nq // qt,),
        in_specs=[pl.BlockSpec((qt, w), lambda i: (i, 0))],
        out_specs=pl.BlockSpec((qt, _NCAND), lambda i: (i, 0)),
        out_shape=jax.ShapeDtypeStruct((nq, _NCAND), jnp.int32),
        compiler_params=pltpu.CompilerParams(
            dimension_semantics=("arbitrary",)),
    )(cm)

    gk = _sc_gather(knp, kid)

    rt = 32
    vals, idx = pl.pallas_call(
        _rescore_body,
        grid=(nq // rt,),
        in_specs=[
            pl.BlockSpec((rt, _NCAND, d), lambda i: (i, 0, 0)),
            pl.BlockSpec((rt, d), lambda i: (i, 0)),
            pl.BlockSpec((rt, _NCAND), lambda i: (i, 0)),
        ],
        out_specs=[
            pl.BlockSpec((rt, 16), lambda i: (i, 0)),
            pl.BlockSpec((rt, 16), lambda i: (i, 0)),
        ],
        out_shape=[
            jax.ShapeDtypeStruct((nq, 16), jnp.float32),
            jax.ShapeDtypeStruct((nq, 16), jnp.int32),
        ],
        compiler_params=pltpu.CompilerParams(
            dimension_semantics=("arbitrary",)),
    )(gk, qn, kid)

    return vals[:, :_TOPK], idx[:, :_TOPK]


# speedup vs baseline: 5.9968x; 1.0551x over previous
import functools

import jax
import jax.numpy as jnp
from jax import lax
from jax.experimental import pallas as pl
from jax.experimental.pallas import tpu as pltpu
from jax.experimental.pallas import tpu_sc as plsc

_B = 2048
_G = 16
_NCAND = 160
_NEG = -3.0e38
_TOPK = 10


def _score_chunkmax_body(n_keys, q_ref, k_ref, cm_ref):
    b = pl.program_id(0)
    kb = k_ref.shape[0]

    qn = q_ref[...]
    kn = k_ref[...]
    s = lax.dot_general(qn, kn, (((1,), (1,)), ((), ())),
                        preferred_element_type=jnp.float32)
    col = lax.broadcasted_iota(jnp.int32, (1, kb), 1) + b * kb
    s = jnp.where(col < n_keys, s, _NEG)

    m = s[:, 0:128]
    for g in range(1, _G):
        m = jnp.maximum(m, s[:, g * 128:(g + 1) * 128])
    cm_ref[...] = m


def _chunk_topk_body(kb, cm_ref, kid_ref):
    c = cm_ref[...]
    t, w = c.shape
    iota_c = lax.broadcasted_iota(jnp.int32, (t, w), 1)
    giota = lax.broadcasted_iota(jnp.int32, (1, _G), 1)
    big = 1 << 30
    for j in range(_TOPK):
        m = jnp.max(c, axis=1, keepdims=True)
        pos = jnp.min(jnp.where(c == m, iota_c, big), axis=1, keepdims=True)
        blk = lax.shift_right_logical(pos, 7)
        lane = pos - blk * 128
        kid16 = blk * kb + lane + giota * 128
        kid_ref[:, j * _G:(j + 1) * _G] = kid16
        c = jnp.where(iota_c == pos, _NEG, c)


def _sc_gather(kn_pad, kid):
    nq = kid.shape[0]
    d = kn_pad.shape[1]
    kid_flat = kid.reshape(nq * _NCAND)
    mesh = plsc.VectorSubcoreMesh(core_axis_name="c", subcore_axis_name="s")
    nw = 32
    qpw = nq // nw
    half = _NCAND // 2

    @functools.partial(
        pl.kernel,
        mesh=mesh,
        out_type=jax.ShapeDtypeStruct((nq, _NCAND, d), jnp.float32),
        scratch_types=[
            pltpu.VMEM((qpw * _NCAND,), jnp.int32),
            pltpu.VMEM((_NCAND, d), jnp.float32),
            pltpu.VMEM((_NCAND, d), jnp.float32),
            pltpu.SemaphoreType.DMA,
        ],
    )
    def gather_kernel(kn_hbm, kid_hbm, out_hbm, idv, rows0, rows1, sem):
        wid = lax.axis_index("s") * 2 + lax.axis_index("c")
        base = wid * qpw
        pltpu.sync_copy(kid_hbm.at[pl.ds(base * _NCAND, qpw * _NCAND)], idv)

        def body(k, carry):
            qa = base + 2 * k
            qb = qa + 1
            oa = 2 * k * _NCAND
            ob = (2 * k + 1) * _NCAND
            d1 = pltpu.async_copy(
                kn_hbm.at[idv.at[pl.ds(oa, half)]], rows0.at[pl.ds(0, half)],
                sem)
            d2 = pltpu.async_copy(
                kn_hbm.at[idv.at[pl.ds(oa + half, half)]],
                rows0.at[pl.ds(half, half)], sem)
            d3 = pltpu.async_copy(
                kn_hbm.at[idv.at[pl.ds(ob, half)]], rows1.at[pl.ds(0, half)],
                sem)
            d4 = pltpu.async_copy(
                kn_hbm.at[idv.at[pl.ds(ob + half, half)]],
                rows1.at[pl.ds(half, half)], sem)
            d1.wait()
            d2.wait()
            d3.wait()
            d4.wait()
            pltpu.sync_copy(rows0, out_hbm.at[qa])
            pltpu.sync_copy(rows1, out_hbm.at[qb])
            return carry

        lax.fori_loop(0, qpw // 2, body, 0)

    return gather_kernel(kn_pad, kid_flat)


def _rescore_body(g_ref, qn_ref, kid_ref, vals_ref, idx_ref):
    t = g_ref.shape[0]
    ncp = 256
    g = g_ref[...]
    gz = jnp.concatenate(
        [g, jnp.zeros((t, ncp - _NCAND, 128), jnp.float32)], axis=1)
    g2 = gz.reshape(t * ncp, 128)
    qn = qn_ref[...]
    full = lax.dot_general(qn, g2, (((1,), (1,)), ((), ())),
                           preferred_element_type=jnp.float32)
    s = jnp.concatenate(
        [full[i:i + 1, i * ncp:(i + 1) * ncp] for i in range(t)], axis=0)
    s = jnp.where(s > 0.0, s, _NEG)
    kid = kid_ref[...]
    iota_nc = lax.broadcasted_iota(jnp.int32, (t, _NCAND), 1)
    iota_p = lax.broadcasted_iota(jnp.int32, (t, ncp), 1)
    big = 1 << 30
    vals_ref[...] = jnp.zeros(vals_ref.shape, jnp.float32)
    idx_ref[...] = jnp.zeros(idx_ref.shape, jnp.int32)
    for j in range(_TOPK):
        m = jnp.max(s, axis=1, keepdims=True)
        pos = jnp.min(jnp.where(s == m, iota_p, big), axis=1, keepdims=True)
        gi = jnp.sum(jnp.where(iota_nc == pos, kid, 0), axis=1, keepdims=True)
        vals_ref[:, j:j + 1] = m
        idx_ref[:, j:j + 1] = gi
        s = jnp.where(iota_p == pos, _NEG, s)


def kernel(queries, keys, top_k):
    nq, d = queries.shape
    n_keys = keys.shape[0]
    kb = _B
    nb = (n_keys + kb - 1) // kb
    kpad = nb * kb
    qn = queries / jnp.linalg.norm(queries, axis=1, keepdims=True)
    kn = keys / jnp.linalg.norm(keys, axis=1, keepdims=True)
    knp = jnp.pad(kn, ((0, kpad - n_keys), (0, 0)))
    w = nb * 128

    cm = pl.pallas_call(
        functools.partial(_score_chunkmax_body, n_keys),
        grid=(nb,),
        in_specs=[
            pl.BlockSpec((nq, d), lambda b: (0, 0)),
            pl.BlockSpec((kb, d), lambda b: (b, 0)),
        ],
        out_specs=pl.BlockSpec((nq, 128), lambda b: (0, b)),
        out_shape=jax.ShapeDtypeStruct((nq, w), jnp.float32),
        compiler_params=pltpu.CompilerParams(
            dimension_semantics=("arbitrary",)),
    )(qn, knp)

    qt = 256
    kid = pl.pallas_call(
        functools.partial(_chunk_topk_body, kb),
        grid=(nq // qt,),
        in_specs=[pl.BlockSpec((qt, w), lambda i: (i, 0))],
        out_specs=pl.BlockSpec((qt, _NCAND), lambda i: (i, 0)),
        out_shape=jax.ShapeDtypeStruct((nq, _NCAND), jnp.int32),
        compiler_params=pltpu.CompilerParams(
            dimension_semantics=("arbitrary",)),
    )(cm)

    gk = _sc_gather(knp, kid)

    rt = 32
    vals, idx = pl.pallas_call(
        _rescore_body,
        grid=(nq // rt,),
        in_specs=[
            pl.BlockSpec((rt, _NCAND, d), lambda i: (i, 0, 0)),
            pl.BlockSpec((rt, d), lambda i: (i, 0)),
            pl.BlockSpec((rt, _NCAND), lambda i: (i, 0)),
        ],
        out_specs=[
            pl.BlockSpec((rt, 16), lambda i: (i, 0)),
            pl.BlockSpec((rt, 16), lambda i: (i, 0)),
        ],
        out_shape=[
            jax.ShapeDtypeStruct((nq, 16), jnp.float32),
            jax.ShapeDtypeStruct((nq, 16), jnp.int32),
        ],
        compiler_params=pltpu.CompilerParams(
            dimension_semantics=("arbitrary",)),
    )(gk, qn, kid)

    return vals[:, :_TOPK], idx[:, :_TOPK]


# speedup vs baseline: 6.2456x; 1.0415x over previous
import functools

import jax
import jax.numpy as jnp
from jax import lax
from jax.experimental import pallas as pl
from jax.experimental.pallas import tpu as pltpu
from jax.experimental.pallas import tpu_sc as plsc

_B = 2048
_G = 16
_NCAND = 160
_NEG = -3.0e38
_TOPK = 10


def _score_chunkmax_body(n_keys, q_ref, k_ref, cm_ref):
    b = pl.program_id(0)
    kb = k_ref.shape[0]

    qn = q_ref[...]
    kn = k_ref[...]
    s = lax.dot_general(qn, kn, (((1,), (1,)), ((), ())),
                        preferred_element_type=jnp.float32)
    col = lax.broadcasted_iota(jnp.int32, (1, kb), 1) + b * kb
    s = jnp.where(col < n_keys, s, _NEG)

    m = s[:, 0:128]
    for g in range(1, _G):
        m = jnp.maximum(m, s[:, g * 128:(g + 1) * 128])
    cm_ref[...] = m


def _chunk_topk_body(kb, cm_ref, kid_ref):
    c = cm_ref[...]
    t, w = c.shape
    iota_c = lax.broadcasted_iota(jnp.int32, (t, w), 1)
    giota = lax.broadcasted_iota(jnp.int32, (1, _G), 1)
    big = 1 << 30
    for j in range(_TOPK):
        m = jnp.max(c, axis=1, keepdims=True)
        pos = jnp.min(jnp.where(c == m, iota_c, big), axis=1, keepdims=True)
        blk = lax.shift_right_logical(pos, 7)
        lane = pos - blk * 128
        kid16 = blk * kb + lane + giota * 128
        kid_ref[:, j * _G:(j + 1) * _G] = kid16
        c = jnp.where(iota_c == pos, _NEG, c)


def _sc_gather(kn_pad, kid):
    nq = kid.shape[0]
    d = kn_pad.shape[1]
    kid_flat = kid.reshape(nq * _NCAND)
    mesh = plsc.VectorSubcoreMesh(core_axis_name="c", subcore_axis_name="s")
    nw = 32
    qpw = nq // nw
    half = _NCAND // 2

    @functools.partial(
        pl.kernel,
        mesh=mesh,
        out_type=jax.ShapeDtypeStruct((nq, _NCAND, d), jnp.float32),
        scratch_types=[
            pltpu.VMEM((qpw * _NCAND,), jnp.int32),
            pltpu.VMEM((_NCAND, d), jnp.float32),
            pltpu.VMEM((_NCAND, d), jnp.float32),
            pltpu.SemaphoreType.DMA,
        ],
    )
    def gather_kernel(kn_hbm, kid_hbm, out_hbm, idv, rows0, rows1, sem):
        wid = lax.axis_index("s") * 2 + lax.axis_index("c")
        base = wid * qpw
        pltpu.sync_copy(kid_hbm.at[pl.ds(base * _NCAND, qpw * _NCAND)], idv)

        def body(k, carry):
            qa = base + 2 * k
            qb = qa + 1
            oa = 2 * k * _NCAND
            ob = (2 * k + 1) * _NCAND
            d1 = pltpu.async_copy(
                kn_hbm.at[idv.at[pl.ds(oa, half)]], rows0.at[pl.ds(0, half)],
                sem)
            d2 = pltpu.async_copy(
                kn_hbm.at[idv.at[pl.ds(oa + half, half)]],
                rows0.at[pl.ds(half, half)], sem)
            d3 = pltpu.async_copy(
                kn_hbm.at[idv.at[pl.ds(ob, half)]], rows1.at[pl.ds(0, half)],
                sem)
            d4 = pltpu.async_copy(
                kn_hbm.at[idv.at[pl.ds(ob + half, half)]],
                rows1.at[pl.ds(half, half)], sem)
            d1.wait()
            d2.wait()
            d3.wait()
            d4.wait()
            pltpu.sync_copy(rows0, out_hbm.at[qa])
            pltpu.sync_copy(rows1, out_hbm.at[qb])
            return carry

        lax.fori_loop(0, qpw // 2, body, 0)

    return gather_kernel(kn_pad, kid_flat)


def _rescore_body(n_keys, g_ref, qn_ref, kid_ref, vals_ref, idx_ref):
    t = g_ref.shape[0]
    ncp = 256
    g = g_ref[...]
    gz = jnp.concatenate(
        [g, jnp.zeros((t, ncp - _NCAND, 128), jnp.float32)], axis=1)
    g2 = gz.reshape(t * ncp, 128)
    qn = qn_ref[...]
    full = lax.dot_general(qn, g2, (((1,), (1,)), ((), ())),
                           preferred_element_type=jnp.float32)
    s = jnp.concatenate(
        [full[i:i + 1, i * ncp:(i + 1) * ncp] for i in range(t)], axis=0)
    kid = kid_ref[...]
    kidp = jnp.concatenate(
        [kid, jnp.full((t, ncp - _NCAND), 1 << 30, jnp.int32)], axis=1)
    s = jnp.where(kidp < n_keys, s, _NEG)
    iota_nc = lax.broadcasted_iota(jnp.int32, (t, _NCAND), 1)
    iota_p = lax.broadcasted_iota(jnp.int32, (t, ncp), 1)
    big = 1 << 30
    vals_ref[...] = jnp.zeros(vals_ref.shape, jnp.float32)
    idx_ref[...] = jnp.zeros(idx_ref.shape, jnp.int32)
    for j in range(_TOPK):
        m = jnp.max(s, axis=1, keepdims=True)
        pos = jnp.min(jnp.where(s == m, iota_p, big), axis=1, keepdims=True)
        gi = jnp.sum(jnp.where(iota_nc == pos, kid, 0), axis=1, keepdims=True)
        vals_ref[:, j:j + 1] = m
        idx_ref[:, j:j + 1] = gi
        s = jnp.where(iota_p == pos, _NEG, s)


def kernel(queries, keys, top_k):
    nq, d = queries.shape
    n_keys = keys.shape[0]
    kb = _B
    nb = (n_keys + kb - 1) // kb
    kpad = nb * kb
    qn = queries / jnp.linalg.norm(queries, axis=1, keepdims=True)
    kn = keys / jnp.linalg.norm(keys, axis=1, keepdims=True)
    w = nb * 128

    cm = pl.pallas_call(
        functools.partial(_score_chunkmax_body, n_keys),
        grid=(nb,),
        in_specs=[
            pl.BlockSpec((nq, d), lambda b: (0, 0)),
            pl.BlockSpec((kb, d), lambda b: (b, 0)),
        ],
        out_specs=pl.BlockSpec((nq, 128), lambda b: (0, b)),
        out_shape=jax.ShapeDtypeStruct((nq, w), jnp.float32),
        compiler_params=pltpu.CompilerParams(
            dimension_semantics=("arbitrary",)),
    )(qn, kn)

    qt = 256
    kid = pl.pallas_call(
        functools.partial(_chunk_topk_body, kb),
        grid=(nq // qt,),
        in_specs=[pl.BlockSpec((qt, w), lambda i: (i, 0))],
        out_specs=pl.BlockSpec((qt, _NCAND), lambda i: (i, 0)),
        out_shape=jax.ShapeDtypeStruct((nq, _NCAND), jnp.int32),
        compiler_params=pltpu.CompilerParams(
            dimension_semantics=("arbitrary",)),
    )(cm)

    gk = _sc_gather(kn, jnp.minimum(kid, n_keys - 1))

    rt = 32
    vals, idx = pl.pallas_call(
        functools.partial(_rescore_body, n_keys),
        grid=(nq // rt,),
        in_specs=[
            pl.BlockSpec((rt, _NCAND, d), lambda i: (i, 0, 0)),
            pl.BlockSpec((rt, d), lambda i: (i, 0)),
            pl.BlockSpec((rt, _NCAND), lambda i: (i, 0)),
        ],
        out_specs=[
            pl.BlockSpec((rt, 16), lambda i: (i, 0)),
            pl.BlockSpec((rt, 16), lambda i: (i, 0)),
        ],
        out_shape=[
            jax.ShapeDtypeStruct((nq, 16), jnp.float32),
            jax.ShapeDtypeStruct((nq, 16), jnp.int32),
        ],
        compiler_params=pltpu.CompilerParams(
            dimension_semantics=("arbitrary",)),
    )(gk, qn, kid)

    return vals[:, :_TOPK], idx[:, :_TOPK]


# speedup vs baseline: 6.2710x; 1.0041x over previous
import functools

import jax
import jax.numpy as jnp
from jax import lax
from jax.experimental import pallas as pl
from jax.experimental.pallas import tpu as pltpu
from jax.experimental.pallas import tpu_sc as plsc

_B = 2048
_G = 16
_NCAND = 160
_NEG = -3.0e38
_TOPK = 10


def _score_chunkmax_body(n_keys, q_ref, k_ref, cm_ref):
    b = pl.program_id(0)
    kb = k_ref.shape[0]

    qn = q_ref[...]
    kn = k_ref[...]
    s = lax.dot_general(qn, kn, (((1,), (1,)), ((), ())),
                        preferred_element_type=jnp.float32)
    col = lax.broadcasted_iota(jnp.int32, (1, kb), 1) + b * kb
    s = jnp.where(col < n_keys, s, _NEG)

    m = s[:, 0:128]
    for g in range(1, _G):
        m = jnp.maximum(m, s[:, g * 128:(g + 1) * 128])
    cm_ref[...] = m


def _chunk_topk_body(kb, cm_ref, kid_ref):
    c = cm_ref[...]
    t, w = c.shape
    iota_c = lax.broadcasted_iota(jnp.int32, (t, w), 1)
    giota = lax.broadcasted_iota(jnp.int32, (1, _G), 1)
    big = 1 << 30
    for j in range(_TOPK):
        m = jnp.max(c, axis=1, keepdims=True)
        pos = jnp.min(jnp.where(c == m, iota_c, big), axis=1, keepdims=True)
        blk = lax.shift_right_logical(pos, 7)
        lane = pos - blk * 128
        kid16 = blk * kb + lane + giota * 128
        kid_ref[:, j * _G:(j + 1) * _G] = kid16
        c = jnp.where(iota_c == pos, _NEG, c)


def _sc_gather(kn_pad, kid):
    nq = kid.shape[0]
    d = kn_pad.shape[1]
    kid_flat = kid.reshape(nq * _NCAND)
    mesh = plsc.VectorSubcoreMesh(core_axis_name="c", subcore_axis_name="s")
    nw = 32
    qpw = nq // nw
    half = _NCAND // 2

    @functools.partial(
        pl.kernel,
        mesh=mesh,
        out_type=jax.ShapeDtypeStruct((nq, _NCAND, d), jnp.float32),
        scratch_types=[
            pltpu.VMEM((qpw * _NCAND,), jnp.int32),
            pltpu.VMEM((_NCAND, d), jnp.float32),
            pltpu.VMEM((_NCAND, d), jnp.float32),
            pltpu.VMEM((_NCAND, d), jnp.float32),
            pltpu.VMEM((_NCAND, d), jnp.float32),
            pltpu.SemaphoreType.DMA,
            pltpu.SemaphoreType.DMA,
        ],
    )
    def gather_kernel(kn_hbm, kid_hbm, out_hbm, idv, rows0, rows1, rows2,
                      rows3, semg, semo):
        wid = lax.axis_index("s") * 2 + lax.axis_index("c")
        base = wid * qpw
        pltpu.sync_copy(kid_hbm.at[pl.ds(base * _NCAND, qpw * _NCAND)], idv)

        def fire(k, ra, rb):
            qa = base + 2 * k
            oa = 2 * k * _NCAND
            ob = oa + _NCAND
            d1 = pltpu.async_copy(
                kn_hbm.at[idv.at[pl.ds(oa, half)]], ra.at[pl.ds(0, half)],
                semg)
            d2 = pltpu.async_copy(
                kn_hbm.at[idv.at[pl.ds(oa + half, half)]],
                ra.at[pl.ds(half, half)], semg)
            d3 = pltpu.async_copy(
                kn_hbm.at[idv.at[pl.ds(ob, half)]], rb.at[pl.ds(0, half)],
                semg)
            d4 = pltpu.async_copy(
                kn_hbm.at[idv.at[pl.ds(ob + half, half)]],
                rb.at[pl.ds(half, half)], semg)
            return d1, d2, d3, d4

        def drain_and_out(k, ra, rb):
            pltpu.async_copy(ra, out_hbm.at[base + 2 * k], semo)
            pltpu.async_copy(rb, out_hbm.at[base + 2 * k + 1], semo)

        def wait_outs(k, ra, rb):
            pltpu.make_async_copy(ra, out_hbm.at[base + 2 * k], semo).wait()
            pltpu.make_async_copy(rb, out_hbm.at[base + 2 * k + 1],
                                  semo).wait()

        def body(k, carry):
            @pl.when(lax.rem(k, 2) == 0)
            def _():
                @pl.when(k > 0)
                def _():
                    wait_outs(k - 2, rows0, rows1)
                ds = fire(k, rows0, rows1)
                for dd in ds:
                    dd.wait()
                drain_and_out(k, rows0, rows1)

            @pl.when(lax.rem(k, 2) == 1)
            def _():
                @pl.when(k > 1)
                def _():
                    wait_outs(k - 2, rows2, rows3)
                ds = fire(k, rows2, rows3)
                for dd in ds:
                    dd.wait()
                drain_and_out(k, rows2, rows3)

            return carry

        npair = qpw // 2
        lax.fori_loop(0, npair, body, 0)
        wait_outs(npair - 2, rows0, rows1)
        wait_outs(npair - 1, rows2, rows3)

    return gather_kernel(kn_pad, kid_flat)


def _rescore_body(n_keys, g_ref, qn_ref, kid_ref, vals_ref, idx_ref):
    t = g_ref.shape[0]
    ncp = 256
    g = g_ref[...]
    gz = jnp.concatenate(
        [g, jnp.zeros((t, ncp - _NCAND, 128), jnp.float32)], axis=1)
    g2 = gz.reshape(t * ncp, 128)
    qn = qn_ref[...]
    full = lax.dot_general(qn, g2, (((1,), (1,)), ((), ())),
                           preferred_element_type=jnp.float32)
    s = jnp.concatenate(
        [full[i:i + 1, i * ncp:(i + 1) * ncp] for i in range(t)], axis=0)
    kid = kid_ref[...]
    kidp = jnp.concatenate(
        [kid, jnp.full((t, ncp - _NCAND), 1 << 30, jnp.int32)], axis=1)
    s = jnp.where(kidp < n_keys, s, _NEG)
    iota_nc = lax.broadcasted_iota(jnp.int32, (t, _NCAND), 1)
    iota_p = lax.broadcasted_iota(jnp.int32, (t, ncp), 1)
    big = 1 << 30
    vals_ref[...] = jnp.zeros(vals_ref.shape, jnp.float32)
    idx_ref[...] = jnp.zeros(idx_ref.shape, jnp.int32)
    for j in range(_TOPK):
        m = jnp.max(s, axis=1, keepdims=True)
        pos = jnp.min(jnp.where(s == m, iota_p, big), axis=1, keepdims=True)
        gi = jnp.sum(jnp.where(iota_nc == pos, kid, 0), axis=1, keepdims=True)
        vals_ref[:, j:j + 1] = m
        idx_ref[:, j:j + 1] = gi
        s = jnp.where(iota_p == pos, _NEG, s)


def kernel(queries, keys, top_k):
    nq, d = queries.shape
    n_keys = keys.shape[0]
    kb = _B
    nb = (n_keys + kb - 1) // kb
    kpad = nb * kb
    qn = queries / jnp.linalg.norm(queries, axis=1, keepdims=True)
    kn = keys / jnp.linalg.norm(keys, axis=1, keepdims=True)
    w = nb * 128

    cm = pl.pallas_call(
        functools.partial(_score_chunkmax_body, n_keys),
        grid=(nb,),
        in_specs=[
            pl.BlockSpec((nq, d), lambda b: (0, 0)),
            pl.BlockSpec((kb, d), lambda b: (b, 0)),
        ],
        out_specs=pl.BlockSpec((nq, 128), lambda b: (0, b)),
        out_shape=jax.ShapeDtypeStruct((nq, w), jnp.float32),
        compiler_params=pltpu.CompilerParams(
            dimension_semantics=("arbitrary",)),
    )(qn, kn)

    qt = 256
    kid = pl.pallas_call(
        functools.partial(_chunk_topk_body, kb),
        grid=(nq // qt,),
        in_specs=[pl.BlockSpec((qt, w), lambda i: (i, 0))],
        out_specs=pl.BlockSpec((qt, _NCAND), lambda i: (i, 0)),
        out_shape=jax.ShapeDtypeStruct((nq, _NCAND), jnp.int32),
        compiler_params=pltpu.CompilerParams(
            dimension_semantics=("arbitrary",)),
    )(cm)

    gk = _sc_gather(kn, jnp.minimum(kid, n_keys - 1))

    rt = 32
    vals, idx = pl.pallas_call(
        functools.partial(_rescore_body, n_keys),
        grid=(nq // rt,),
        in_specs=[
            pl.BlockSpec((rt, _NCAND, d), lambda i: (i, 0, 0)),
            pl.BlockSpec((rt, d), lambda i: (i, 0)),
            pl.BlockSpec((rt, _NCAND), lambda i: (i, 0)),
        ],
        out_specs=[
            pl.BlockSpec((rt, 16), lambda i: (i, 0)),
            pl.BlockSpec((rt, 16), lambda i: (i, 0)),
        ],
        out_shape=[
            jax.ShapeDtypeStruct((nq, 16), jnp.float32),
            jax.ShapeDtypeStruct((nq, 16), jnp.int32),
        ],
        compiler_params=pltpu.CompilerParams(
            dimension_semantics=("arbitrary",)),
    )(gk, qn, kid)

    return vals[:, :_TOPK], idx[:, :_TOPK]


# speedup vs baseline: 6.2780x; 1.0011x over previous
import functools

import jax
import jax.numpy as jnp
from jax import lax
from jax.experimental import pallas as pl
from jax.experimental.pallas import tpu as pltpu
from jax.experimental.pallas import tpu_sc as plsc

_B = 2048
_G = 16
_NCAND = 160
_NEG = -3.0e38
_TOPK = 10


def _score_select_body(nb, n_keys, qt2, q_ref, k_ref, kid_ref, cm_scr):
    b = pl.program_id(0)
    kb = k_ref.shape[0]

    @pl.when(b < nb)
    def _score():
        qn = q_ref[...]
        kn = k_ref[...]
        s = lax.dot_general(qn, kn, (((1,), (1,)), ((), ())),
                            preferred_element_type=jnp.float32)
        col = lax.broadcasted_iota(jnp.int32, (1, kb), 1) + b * kb
        s = jnp.where(col < n_keys, s, _NEG)
        m = s[:, 0:128]
        for g in range(1, _G):
            m = jnp.maximum(m, s[:, g * 128:(g + 1) * 128])
        cm_scr[b] = m

    @pl.when(b >= nb)
    def _select():
        t = b - nb
        s = cm_scr[:, pl.ds(t * qt2, qt2), :]
        bi = lax.broadcasted_iota(jnp.int32, (nb, qt2, 128), 0)
        li = lax.broadcasted_iota(jnp.int32, (nb, qt2, 128), 2)
        cid = bi * 128 + li
        giota = lax.broadcasted_iota(jnp.int32, (1, _G), 1)
        big = 1 << 30
        for j in range(_TOPK):
            m = jnp.max(jnp.max(s, axis=0), axis=1, keepdims=True)
            m3 = m.reshape(1, qt2, 1)
            pos = jnp.min(jnp.min(jnp.where(s == m3, cid, big), axis=0),
                          axis=1, keepdims=True)
            blk = lax.shift_right_logical(pos, 7)
            lane = pos - blk * 128
            kid16 = blk * kb + lane + giota * 128
            kid_ref[:, j * _G:(j + 1) * _G] = kid16
            s = jnp.where(cid == pos.reshape(1, qt2, 1), _NEG, s)


def _sc_gather(kn_pad, kid):
    nq = kid.shape[0]
    d = kn_pad.shape[1]
    kid_flat = kid.reshape(nq * _NCAND)
    mesh = plsc.VectorSubcoreMesh(core_axis_name="c", subcore_axis_name="s")
    nw = 32
    qpw = nq // nw
    half = _NCAND // 2

    @functools.partial(
        pl.kernel,
        mesh=mesh,
        out_type=jax.ShapeDtypeStruct((nq, _NCAND, d), jnp.float32),
        scratch_types=[
            pltpu.VMEM((qpw * _NCAND,), jnp.int32),
            pltpu.VMEM((_NCAND, d), jnp.float32),
            pltpu.VMEM((_NCAND, d), jnp.float32),
            pltpu.VMEM((_NCAND, d), jnp.float32),
            pltpu.VMEM((_NCAND, d), jnp.float32),
            pltpu.SemaphoreType.DMA,
            pltpu.SemaphoreType.DMA,
        ],
    )
    def gather_kernel(kn_hbm, kid_hbm, out_hbm, idv, rows0, rows1, rows2,
                      rows3, semg, semo):
        wid = lax.axis_index("s") * 2 + lax.axis_index("c")
        base = wid * qpw
        pltpu.sync_copy(kid_hbm.at[pl.ds(base * _NCAND, qpw * _NCAND)], idv)

        def fire(k, ra, rb):
            qa = base + 2 * k
            oa = 2 * k * _NCAND
            ob = oa + _NCAND
            d1 = pltpu.async_copy(
                kn_hbm.at[idv.at[pl.ds(oa, half)]], ra.at[pl.ds(0, half)],
                semg)
            d2 = pltpu.async_copy(
                kn_hbm.at[idv.at[pl.ds(oa + half, half)]],
                ra.at[pl.ds(half, half)], semg)
            d3 = pltpu.async_copy(
                kn_hbm.at[idv.at[pl.ds(ob, half)]], rb.at[pl.ds(0, half)],
                semg)
            d4 = pltpu.async_copy(
                kn_hbm.at[idv.at[pl.ds(ob + half, half)]],
                rb.at[pl.ds(half, half)], semg)
            return d1, d2, d3, d4

        def drain_and_out(k, ra, rb):
            pltpu.async_copy(ra, out_hbm.at[base + 2 * k], semo)
            pltpu.async_copy(rb, out_hbm.at[base + 2 * k + 1], semo)

        def wait_outs(k, ra, rb):
            pltpu.make_async_copy(ra, out_hbm.at[base + 2 * k], semo).wait()
            pltpu.make_async_copy(rb, out_hbm.at[base + 2 * k + 1],
                                  semo).wait()

        def body(k, carry):
            @pl.when(lax.rem(k, 2) == 0)
            def _():
                @pl.when(k > 0)
                def _():
                    wait_outs(k - 2, rows0, rows1)
                ds = fire(k, rows0, rows1)
                for dd in ds:
                    dd.wait()
                drain_and_out(k, rows0, rows1)

            @pl.when(lax.rem(k, 2) == 1)
            def _():
                @pl.when(k > 1)
                def _():
                    wait_outs(k - 2, rows2, rows3)
                ds = fire(k, rows2, rows3)
                for dd in ds:
                    dd.wait()
                drain_and_out(k, rows2, rows3)

            return carry

        npair = qpw // 2
        lax.fori_loop(0, npair, body, 0)
        wait_outs(npair - 2, rows0, rows1)
        wait_outs(npair - 1, rows2, rows3)

    return gather_kernel(kn_pad, kid_flat)


def _rescore_body(n_keys, g_ref, qn_ref, kid_ref, vals_ref, idx_ref):
    t = g_ref.shape[0]
    ncp = 256
    g = g_ref[...]
    gz = jnp.concatenate(
        [g, jnp.zeros((t, ncp - _NCAND, 128), jnp.float32)], axis=1)
    g2 = gz.reshape(t * ncp, 128)
    qn = qn_ref[...]
    full = lax.dot_general(qn, g2, (((1,), (1,)), ((), ())),
                           preferred_element_type=jnp.float32)
    s = jnp.concatenate(
        [full[i:i + 1, i * ncp:(i + 1) * ncp] for i in range(t)], axis=0)
    kid = kid_ref[...]
    kidp = jnp.concatenate(
        [kid, jnp.full((t, ncp - _NCAND), 1 << 30, jnp.int32)], axis=1)
    s = jnp.where(kidp < n_keys, s, _NEG)
    iota_nc = lax.broadcasted_iota(jnp.int32, (t, _NCAND), 1)
    iota_p = lax.broadcasted_iota(jnp.int32, (t, ncp), 1)
    big = 1 << 30
    vals_ref[...] = jnp.zeros(vals_ref.shape, jnp.float32)
    idx_ref[...] = jnp.zeros(idx_ref.shape, jnp.int32)
    for j in range(_TOPK):
        m = jnp.max(s, axis=1, keepdims=True)
        pos = jnp.min(jnp.where(s == m, iota_p, big), axis=1, keepdims=True)
        gi = jnp.sum(jnp.where(iota_nc == pos, kid, 0), axis=1, keepdims=True)
        vals_ref[:, j:j + 1] = m
        idx_ref[:, j:j + 1] = gi
        s = jnp.where(iota_p == pos, _NEG, s)


def kernel(queries, keys, top_k):
    nq, d = queries.shape
    n_keys = keys.shape[0]
    kb = _B
    nb = (n_keys + kb - 1) // kb
    kpad = nb * kb
    qn = queries / jnp.linalg.norm(queries, axis=1, keepdims=True)
    kn = keys / jnp.linalg.norm(keys, axis=1, keepdims=True)
    qt2 = 128
    kid = pl.pallas_call(
        functools.partial(_score_select_body, nb, n_keys, qt2),
        grid=(nb + nq // qt2,),
        in_specs=[
            pl.BlockSpec((nq, d), lambda b: (0, 0)),
            pl.BlockSpec((kb, d), lambda b: (jnp.minimum(b, nb - 1), 0)),
        ],
        out_specs=pl.BlockSpec(
            (qt2, _NCAND), lambda b: (jnp.maximum(b - nb, 0), 0)),
        out_shape=jax.ShapeDtypeStruct((nq, _NCAND), jnp.int32),
        scratch_shapes=[pltpu.VMEM((nb, nq, 128), jnp.float32)],
        compiler_params=pltpu.CompilerParams(
            dimension_semantics=("arbitrary",)),
    )(qn, kn)

    gk = _sc_gather(kn, jnp.minimum(kid, n_keys - 1))

    rt = 32
    vals, idx = pl.pallas_call(
        functools.partial(_rescore_body, n_keys),
        grid=(nq // rt,),
        in_specs=[
            pl.BlockSpec((rt, _NCAND, d), lambda i: (i, 0, 0)),
            pl.BlockSpec((rt, d), lambda i: (i, 0)),
            pl.BlockSpec((rt, _NCAND), lambda i: (i, 0)),
        ],
        out_specs=[
            pl.BlockSpec((rt, 16), lambda i: (i, 0)),
            pl.BlockSpec((rt, 16), lambda i: (i, 0)),
        ],
        out_shape=[
            jax.ShapeDtypeStruct((nq, 16), jnp.float32),
            jax.ShapeDtypeStruct((nq, 16), jnp.int32),
        ],
        compiler_params=pltpu.CompilerParams(
            dimension_semantics=("arbitrary",)),
    )(gk, qn, kid)

    return vals[:, :_TOPK], idx[:, :_TOPK]


# speedup vs baseline: 6.2975x; 1.0031x over previous
import functools

import jax
import jax.numpy as jnp
from jax import lax
from jax.experimental import pallas as pl
from jax.experimental.pallas import tpu as pltpu
from jax.experimental.pallas import tpu_sc as plsc

_B = 2048
_G = 16
_NCAND = 160
_NEG = -3.0e38
_TOPK = 10


def _score_select_body(nb, n_keys, qt2, q_ref, k_ref, kid_ref, cm_scr):
    b = pl.program_id(0)
    kb = k_ref.shape[0]

    @pl.when(b < nb)
    def _score():
        qn = q_ref[...]
        kn = k_ref[...]
        s = lax.dot_general(qn, kn, (((1,), (1,)), ((), ())),
                            preferred_element_type=jnp.float32)
        col = lax.broadcasted_iota(jnp.int32, (1, kb), 1) + b * kb
        s = jnp.where(col < n_keys, s, _NEG)
        m = s[:, 0:128]
        for g in range(1, _G):
            m = jnp.maximum(m, s[:, g * 128:(g + 1) * 128])
        cm_scr[b] = m

    @pl.when(b >= nb)
    def _select():
        t = b - nb
        s = cm_scr[:, pl.ds(t * qt2, qt2), :]
        bi = lax.broadcasted_iota(jnp.int32, (nb, qt2, 128), 0)
        li = lax.broadcasted_iota(jnp.int32, (nb, qt2, 128), 2)
        cid = bi * 128 + li
        giota = lax.broadcasted_iota(jnp.int32, (1, _G), 1)
        big = 1 << 30
        for j in range(_TOPK):
            m = jnp.max(jnp.max(s, axis=0), axis=1, keepdims=True)
            m3 = m.reshape(1, qt2, 1)
            pos = jnp.min(jnp.min(jnp.where(s == m3, cid, big), axis=0),
                          axis=1, keepdims=True)
            blk = lax.shift_right_logical(pos, 7)
            lane = pos - blk * 128
            kid16 = blk * kb + lane + giota * 128
            kid_ref[:, j * _G:(j + 1) * _G] = kid16
            s = jnp.where(cid == pos.reshape(1, qt2, 1), _NEG, s)


def _sc_gather(kn_pad, kid):
    nq = kid.shape[0]
    d = kn_pad.shape[1]
    kid_flat = kid.reshape(nq * _NCAND)
    mesh = plsc.VectorSubcoreMesh(core_axis_name="c", subcore_axis_name="s")
    nw = 32
    qpw = nq // nw
    half = _NCAND // 2

    @functools.partial(
        pl.kernel,
        mesh=mesh,
        out_type=jax.ShapeDtypeStruct((nq, _NCAND, d), jnp.float32),
        scratch_types=[
            pltpu.VMEM((qpw * _NCAND,), jnp.int32),
            pltpu.VMEM((_NCAND, d), jnp.float32),
            pltpu.VMEM((_NCAND, d), jnp.float32),
            pltpu.VMEM((_NCAND, d), jnp.float32),
            pltpu.VMEM((_NCAND, d), jnp.float32),
            pltpu.SemaphoreType.DMA,
            pltpu.SemaphoreType.DMA,
        ],
    )
    def gather_kernel(kn_hbm, kid_hbm, out_hbm, idv, rows0, rows1, rows2,
                      rows3, semg, semo):
        wid = lax.axis_index("s") * 2 + lax.axis_index("c")
        base = wid * qpw
        pltpu.sync_copy(kid_hbm.at[pl.ds(base * _NCAND, qpw * _NCAND)], idv)

        def fire(k, ra, rb):
            qa = base + 2 * k
            oa = 2 * k * _NCAND
            ob = oa + _NCAND
            d1 = pltpu.async_copy(
                kn_hbm.at[idv.at[pl.ds(oa, half)]], ra.at[pl.ds(0, half)],
                semg)
            d2 = pltpu.async_copy(
                kn_hbm.at[idv.at[pl.ds(oa + half, half)]],
                ra.at[pl.ds(half, half)], semg)
            d3 = pltpu.async_copy(
                kn_hbm.at[idv.at[pl.ds(ob, half)]], rb.at[pl.ds(0, half)],
                semg)
            d4 = pltpu.async_copy(
                kn_hbm.at[idv.at[pl.ds(ob + half, half)]],
                rb.at[pl.ds(half, half)], semg)
            return d1, d2, d3, d4

        def drain_and_out(k, ra, rb):
            pltpu.async_copy(ra, out_hbm.at[base + 2 * k], semo)
            pltpu.async_copy(rb, out_hbm.at[base + 2 * k + 1], semo)

        def wait_outs(k, ra, rb):
            pltpu.make_async_copy(ra, out_hbm.at[base + 2 * k], semo).wait()
            pltpu.make_async_copy(rb, out_hbm.at[base + 2 * k + 1],
                                  semo).wait()

        def body(k, carry):
            @pl.when(lax.rem(k, 2) == 0)
            def _():
                @pl.when(k > 0)
                def _():
                    wait_outs(k - 2, rows0, rows1)
                ds = fire(k, rows0, rows1)
                for dd in ds:
                    dd.wait()
                drain_and_out(k, rows0, rows1)

            @pl.when(lax.rem(k, 2) == 1)
            def _():
                @pl.when(k > 1)
                def _():
                    wait_outs(k - 2, rows2, rows3)
                ds = fire(k, rows2, rows3)
                for dd in ds:
                    dd.wait()
                drain_and_out(k, rows2, rows3)

            return carry

        npair = qpw // 2
        lax.fori_loop(0, npair, body, 0)
        wait_outs(npair - 2, rows0, rows1)
        wait_outs(npair - 1, rows2, rows3)

    return gather_kernel(kn_pad, kid_flat)


def _rescore_body(n_keys, g_ref, qn_ref, kid_ref, vals_ref, idx_ref):
    t = g_ref.shape[0]
    ncp = 256
    g = g_ref[...]
    gz = jnp.concatenate(
        [g, jnp.zeros((t, ncp - _NCAND, 128), jnp.float32)], axis=1)
    g2 = gz.reshape(t * ncp, 128)
    qn = qn_ref[...]
    full = lax.dot_general(qn, g2, (((1,), (1,)), ((), ())),
                           preferred_element_type=jnp.float32)
    s = jnp.concatenate(
        [full[i:i + 1, i * ncp:(i + 1) * ncp] for i in range(t)], axis=0)
    kid = kid_ref[...]
    kidp = jnp.concatenate(
        [kid, jnp.full((t, ncp - _NCAND), 1 << 30, jnp.int32)], axis=1)
    s = jnp.where(kidp < n_keys, s, _NEG)
    iota_nc = lax.broadcasted_iota(jnp.int32, (t, _NCAND), 1)
    iota_p = lax.broadcasted_iota(jnp.int32, (t, ncp), 1)
    big = 1 << 30
    vals_ref[...] = jnp.zeros(vals_ref.shape, jnp.float32)
    idx_ref[...] = jnp.zeros(idx_ref.shape, jnp.int32)
    for j in range(_TOPK):
        m = jnp.max(s, axis=1, keepdims=True)
        pos = jnp.min(jnp.where(s == m, iota_p, big), axis=1, keepdims=True)
        gi = jnp.sum(jnp.where(iota_nc == pos, kid, 0), axis=1, keepdims=True)
        vals_ref[:, j:j + 1] = m
        idx_ref[:, j:j + 1] = gi
        s = jnp.where(iota_p == pos, _NEG, s)


def kernel(queries, keys, top_k):
    nq, d = queries.shape
    n_keys = keys.shape[0]
    kb = _B
    nb = (n_keys + kb - 1) // kb
    qn = queries / jnp.linalg.norm(queries, axis=1, keepdims=True)
    kn = keys / jnp.linalg.norm(keys, axis=1, keepdims=True)
    qt2 = 128
    kid = pl.pallas_call(
        functools.partial(_score_select_body, nb, n_keys, qt2),
        grid=(nb + nq // qt2,),
        in_specs=[
            pl.BlockSpec((nq, d), lambda b: (0, 0)),
            pl.BlockSpec((kb, d), lambda b: (jnp.minimum(b, nb - 1), 0)),
        ],
        out_specs=pl.BlockSpec(
            (qt2, _NCAND), lambda b: (jnp.maximum(b - nb, 0), 0)),
        out_shape=jax.ShapeDtypeStruct((nq, _NCAND), jnp.int32),
        scratch_shapes=[pltpu.VMEM((nb, nq, 128), jnp.float32)],
        compiler_params=pltpu.CompilerParams(
            dimension_semantics=("arbitrary",)),
    )(qn, kn)

    gk = _sc_gather(kn, jnp.minimum(kid, n_keys - 1))

    rt = 32
    vals, idx = pl.pallas_call(
        functools.partial(_rescore_body, n_keys),
        grid=(nq // rt,),
        in_specs=[
            pl.BlockSpec((rt, _NCAND, d), lambda i: (i, 0, 0)),
            pl.BlockSpec((rt, d), lambda i: (i, 0)),
            pl.BlockSpec((rt, _NCAND), lambda i: (i, 0)),
        ],
        out_specs=[
            pl.BlockSpec((rt, 16), lambda i: (i, 0)),
            pl.BlockSpec((rt, 16), lambda i: (i, 0)),
        ],
        out_shape=[
            jax.ShapeDtypeStruct((nq, 16), jnp.float32),
            jax.ShapeDtypeStruct((nq, 16), jnp.int32),
        ],
        compiler_params=pltpu.CompilerParams(
            dimension_semantics=("arbitrary",)),
    )(gk, qn, kid)

    return vals[:, :_TOPK], idx[:, :_TOPK]
